# baseline (speedup 1.0000x reference)
import functools

import jax
import jax.numpy as jnp
from jax import lax
from jax.experimental import pallas as pl
from jax.experimental.pallas import tpu as pltpu
from jax.experimental.pallas import tpu_sc as plsc

_NC = 2
_NS = 16
_NW = _NC * _NS
_ICHUNK = 128


def _dist_argmin_body(x_ref, w_ref, xsq_ref, esq_ref, idx_ref):
    s = lax.dot_general(
        x_ref[...], w_ref[...],
        (((1,), (0,)), ((), ())),
        preferred_element_type=jnp.float32,
    )
    dist = (xsq_ref[...] - 2.0 * s) + esq_ref[...]
    idx_ref[0, 0, :] = jnp.argmin(dist, axis=1).astype(jnp.int32)


def _argmin_indices(x_flat, weight, x_sq, e_sq, blk):
    n, d = x_flat.shape
    k = weight.shape[1]
    grid = n // blk
    idx3 = pl.pallas_call(
        _dist_argmin_body,
        grid=(grid,),
        in_specs=[
            pl.BlockSpec((blk, d), lambda i: (i, 0)),
            pl.BlockSpec((d, k), lambda i: (0, 0)),
            pl.BlockSpec((blk, 1), lambda i: (i, 0)),
            pl.BlockSpec((1, k), lambda i: (0, 0)),
        ],
        out_specs=pl.BlockSpec((1, 1, blk), lambda i: (i, 0, 0)),
        out_shape=jax.ShapeDtypeStruct((grid, 1, blk), jnp.int32),
    )(x_flat, weight, x_sq, e_sq)
    return idx3.reshape(n)


def _sc_gather(table, idx, n, d):
    bpw = n // _NW
    nchunk = bpw // _ICHUNK
    idx3 = idx.reshape(_NW, nchunk, _ICHUNK)
    mesh = plsc.VectorSubcoreMesh(core_axis_name="c", subcore_axis_name="s")

    @functools.partial(
        pl.kernel,
        mesh=mesh,
        out_type=jax.ShapeDtypeStruct((_NW, nchunk, _ICHUNK, d), jnp.float32),
        scratch_types=[
            pltpu.VMEM((nchunk, _ICHUNK), jnp.int32),
            pltpu.VMEM((nchunk, _ICHUNK, d), jnp.float32),
            pltpu.SemaphoreType.DMA,
        ],
        compiler_params=pltpu.CompilerParams(use_tc_tiling_on_sc=False),
    )
    def gather_kernel(table_hbm, idx_hbm, out_hbm, idx_v, rows_v, sem):
        wid = lax.axis_index("s") * _NC + lax.axis_index("c")
        pltpu.sync_copy(idx_hbm.at[wid], idx_v)
        copies = [
            pltpu.async_copy(table_hbm.at[idx_v.at[j]], rows_v.at[j], sem)
            for j in range(nchunk)
        ]
        for c in copies:
            c.wait()
        pltpu.sync_copy(rows_v, out_hbm.at[wid])

    return gather_kernel(table, idx3).reshape(n, d)


def kernel(x, weight):
    b, d, h, w = x.shape
    k = weight.shape[1]
    n = b * h * w
    x_flat = jnp.moveaxis(x, 1, -1).reshape(-1, d)
    emb_t = weight.T
    x_sq = jnp.sum(x_flat * x_flat, axis=1, keepdims=True)
    e_sq = jnp.sum(emb_t * emb_t, axis=1)[None, :]
    idx = _argmin_indices(x_flat, weight, x_sq, e_sq, blk=256)
    result_flat = _sc_gather(emb_t, idx, n, d)
    result = jnp.moveaxis(result_flat.reshape(b, h, w, d), -1, 1)
    return result, idx.reshape(b, h, w)


# speedup vs baseline: 1.1520x; 1.1520x over previous
import functools

import jax
import jax.numpy as jnp
from jax import lax
from jax.experimental import pallas as pl
from jax.experimental.pallas import tpu as pltpu
from jax.experimental.pallas import tpu_sc as plsc

_NC = 2
_NS = 16
_NW = _NC * _NS
_ICHUNK = 128


def _dist_argmin_body(x_ref, w_ref, xsq_ref, esq_ref, idx_ref):
    s = lax.dot_general(
        x_ref[0], w_ref[...],
        (((0,), (0,)), ((), ())),
        preferred_element_type=jnp.float32,
    )
    dist = (xsq_ref[0] - 2.0 * s) + esq_ref[...]
    idx_ref[0, 0, :] = jnp.argmin(dist, axis=1).astype(jnp.int32)


def _argmin_indices(x3, weight, x_sq, e_sq):
    b, d, m = x3.shape
    k = weight.shape[1]
    idx3 = pl.pallas_call(
        _dist_argmin_body,
        grid=(b,),
        in_specs=[
            pl.BlockSpec((1, d, m), lambda i: (i, 0, 0)),
            pl.BlockSpec((d, k), lambda i: (0, 0)),
            pl.BlockSpec((1, m, 1), lambda i: (i, 0, 0)),
            pl.BlockSpec((1, k), lambda i: (0, 0)),
        ],
        out_specs=pl.BlockSpec((1, 1, m), lambda i: (i, 0, 0)),
        out_shape=jax.ShapeDtypeStruct((b, 1, m), jnp.int32),
    )(x3, weight, x_sq, e_sq)
    return idx3.reshape(b * m)


def _sc_gather(table, idx, n, d):
    bpw = n // _NW
    nchunk = bpw // _ICHUNK
    idx3 = idx.reshape(_NW, nchunk, _ICHUNK)
    mesh = plsc.VectorSubcoreMesh(core_axis_name="c", subcore_axis_name="s")

    @functools.partial(
        pl.kernel,
        mesh=mesh,
        out_type=jax.ShapeDtypeStruct((_NW, nchunk, _ICHUNK, d), jnp.float32),
        scratch_types=[
            pltpu.VMEM((nchunk, _ICHUNK), jnp.int32),
            pltpu.VMEM((nchunk, _ICHUNK, d), jnp.float32),
            pltpu.SemaphoreType.DMA,
        ],
        compiler_params=pltpu.CompilerParams(use_tc_tiling_on_sc=False),
    )
    def gather_kernel(table_hbm, idx_hbm, out_hbm, idx_v, rows_v, sem):
        wid = lax.axis_index("s") * _NC + lax.axis_index("c")
        pltpu.sync_copy(idx_hbm.at[wid], idx_v)
        copies = [
            pltpu.async_copy(table_hbm.at[idx_v.at[j]], rows_v.at[j], sem)
            for j in range(nchunk)
        ]
        for c in copies:
            c.wait()
        pltpu.sync_copy(rows_v, out_hbm.at[wid])

    return gather_kernel(table, idx3).reshape(n, d)


def kernel(x, weight):
    b, d, h, w = x.shape
    k = weight.shape[1]
    n = b * h * w
    x_flat = jnp.moveaxis(x, 1, -1).reshape(-1, d)
    emb_t = weight.T
    x_sq = jnp.sum(x_flat * x_flat, axis=1, keepdims=True)
    e_sq = jnp.sum(emb_t * emb_t, axis=1)[None, :]
    x3 = x.reshape(b, d, h * w)
    idx = _argmin_indices(x3, weight, x_sq.reshape(b, h * w, 1), e_sq)
    result_flat = _sc_gather(emb_t, idx, n, d)
    result = jnp.moveaxis(result_flat.reshape(b, h, w, d), -1, 1)
    return result, idx.reshape(b, h, w)


# speedup vs baseline: 1.2395x; 1.0759x over previous
import functools

import jax
import jax.numpy as jnp
from jax import lax
from jax.experimental import pallas as pl
from jax.experimental.pallas import tpu as pltpu
from jax.experimental.pallas import tpu_sc as plsc

_NC = 2
_NS = 16
_NW = _NC * _NS
_ICHUNK = 128


def _dist_argmin_body(x_ref, w_ref, xsq_ref, esq_ref, idx_ref):
    s = lax.dot_general(
        x_ref[0], w_ref[...],
        (((0,), (0,)), ((), ())),
        preferred_element_type=jnp.float32,
    )
    dist = (xsq_ref[0] - 2.0 * s) + esq_ref[...]
    idx_ref[0, 0, :] = jnp.argmin(dist, axis=1).astype(jnp.int32)


def _argmin_indices(x3, weight, x_sq, e_sq):
    b, d, m = x3.shape
    k = weight.shape[1]
    idx3 = pl.pallas_call(
        _dist_argmin_body,
        grid=(b,),
        in_specs=[
            pl.BlockSpec((1, d, m), lambda i: (i, 0, 0)),
            pl.BlockSpec((d, k), lambda i: (0, 0)),
            pl.BlockSpec((1, m, 1), lambda i: (i, 0, 0)),
            pl.BlockSpec((1, k), lambda i: (0, 0)),
        ],
        out_specs=pl.BlockSpec((1, 1, m), lambda i: (i, 0, 0)),
        out_shape=jax.ShapeDtypeStruct((b, 1, m), jnp.int32),
    )(x3, weight, x_sq, e_sq)
    return idx3.reshape(b * m)


def _sc_gather(table, idx, n, d):
    bpw = n // _NW
    nchunk = bpw // _ICHUNK
    idx3 = idx.reshape(_NW, nchunk, _ICHUNK)
    mesh = plsc.VectorSubcoreMesh(core_axis_name="c", subcore_axis_name="s")

    @functools.partial(
        pl.kernel,
        mesh=mesh,
        out_type=jax.ShapeDtypeStruct((_NW, nchunk, _ICHUNK, d), jnp.float32),
        scratch_types=[
            pltpu.VMEM((nchunk, _ICHUNK), jnp.int32),
            pltpu.VMEM((nchunk, _ICHUNK, d), jnp.float32),
            pltpu.SemaphoreType.DMA,
        ],
        compiler_params=pltpu.CompilerParams(use_tc_tiling_on_sc=False),
    )
    def gather_kernel(table_hbm, idx_hbm, out_hbm, idx_v, rows_v, sem):
        wid = lax.axis_index("s") * _NC + lax.axis_index("c")
        pltpu.sync_copy(idx_hbm.at[wid], idx_v)
        copies = [
            pltpu.async_copy(table_hbm.at[idx_v.at[j]], rows_v.at[j], sem)
            for j in range(nchunk)
        ]
        for c in copies:
            c.wait()
        pltpu.sync_copy(rows_v, out_hbm.at[wid])

    return gather_kernel(table, idx3).reshape(n, d)


def kernel(x, weight):
    b, d, h, w = x.shape
    k = weight.shape[1]
    n = b * h * w
    x_flat = jnp.moveaxis(x, 1, -1).reshape(-1, d)
    emb_t = weight.T
    x_sq = jnp.sum(x_flat * x_flat, axis=1, keepdims=True)
    e_sq = jnp.sum(emb_t * emb_t, axis=1)[None, :]
    x3 = x.reshape(b, d, h * w)
    idx = _argmin_indices(x3, weight, x_sq.reshape(b, h * w, 1), e_sq)
    result_flat = jnp.take(emb_t, idx, axis=0)
    result = jnp.moveaxis(result_flat.reshape(b, h, w, d), -1, 1)
    return result, idx.reshape(b, h, w)
